# baseline (speedup 1.0000x reference)
import functools

import jax
import jax.numpy as jnp
from jax.experimental import pallas as pl
from jax.experimental.pallas import tpu as pltpu

_ALPHA = 0.1
_BETA = 0.5
_BM = 400
_G = 2000
_W = 2560
_F8 = jnp.float8_e4m3fn
_S2_SCALE = 1.0 / 64.0
_PAD = 8


def _mlp_kernel(x_ref, w_ref, b_ref, o_ref):
    h = jnp.dot(x_ref[...].astype(jnp.bfloat16),
                w_ref[...].astype(jnp.bfloat16),
                preferred_element_type=jnp.float32)
    o_ref[...] = jnp.maximum(h + b_ref[...], 0.0).astype(jnp.bfloat16)


def _layer_kernel(adj_ref, sup_ref, h_blk_ref, wc_ref,
                  o_ref, s8_ref, d_ref, p2_ref, *rest,
                  nch, pubs):
    a8_refs = rest[:nch]
    s2vis_ref, stage_ref = rest[nch], rest[nch + 1]
    i = pl.program_id(0)

    @pl.when(i == 0)
    def _():
        s2vis_ref[...] = jnp.zeros_like(s2vis_ref)

    for step, start, size in pubs:
        @pl.when(i == step)
        def _(start=start, size=size):
            s2vis_ref[start:start + size, :] = stage_ref[start:start + size, :]

    a = adj_ref[...]
    a8 = a.astype(_F8)
    n = a.shape[1]
    for k, a8_ref in enumerate(a8_refs):
        lo = k * _W
        hi = min(lo + _W, n)
        piece = a8[:, lo:hi]
        if hi - lo < _W:
            piece = jnp.concatenate(
                [piece, jnp.zeros((a.shape[0], _W - (hi - lo)), _F8)], axis=1)
        a8_ref[...] = piece

    ab = a.astype(jnp.bfloat16)
    agg = jnp.dot(ab, sup_ref[...], preferred_element_type=jnp.float32)
    p2_ref[...] = jnp.dot(ab, s2vis_ref[...],
                          preferred_element_type=jnp.float32)
    mix = jnp.dot(h_blk_ref[...], wc_ref[...].astype(jnp.bfloat16),
                  preferred_element_type=jnp.float32)
    out = jnp.maximum((1.0 - _BETA) * agg + _BETA * mix, 0.0)
    s2 = ((1.0 - _ALPHA) * out
          + _ALPHA * h_blk_ref[...].astype(jnp.float32))
    s2_bf = s2.astype(jnp.bfloat16)
    o_ref[...] = s2_bf
    stage_ref[pl.ds(i * _BM, _BM), :] = s2_bf

    s8 = (s2 * _S2_SCALE).astype(_F8)
    probe = jnp.where(
        jax.lax.broadcasted_iota(jnp.int32, (s2.shape[0], _PAD), 1) == 0,
        jnp.float32(_S2_SCALE), 0.0).astype(_F8)
    s8_ref[...] = jnp.concatenate([s8, probe], axis=1)

    delta = s2 - s8.astype(jnp.float32) * (1.0 / _S2_SCALE)
    dcol = jnp.sum(delta, axis=0, keepdims=True)[None]

    @pl.when(i == 0)
    def _():
        d_ref[...] = dcol

    @pl.when(i > 0)
    def _():
        d_ref[...] += dcol


def _final_kernel(*refs, nch, n, lens, kmins):
    a8_refs = refs[:nch]
    (sup8_ref, s_blk_ref, d_ref, p2_ref, wc_ref, w2_ref, b2_ref,
     o_ref, acc_ref) = refs[nch:]
    f = s_blk_ref.shape[1]
    t = pl.program_id(0)

    i2 = jnp.int32(0)
    cum = 0
    for g in range(len(lens) - 1):
        cum += lens[g]
        i2 = i2 + (t >= cum).astype(jnp.int32)
    tstart = jnp.int32(0)
    cum = 0
    for g in range(1, len(lens)):
        cum += lens[g - 1]
        tstart = tstart + (i2 >= g).astype(jnp.int32) * lens[g - 1]
    j2 = (nch - 1) - (t - tstart)
    kmin = jnp.int32(0)
    for g in range(1, len(lens)):
        kmin = kmin + (i2 >= g).astype(jnp.int32) * (kmins[g] - kmins[g - 1])

    @pl.when(j2 == nch - 1)
    def _():
        acc_ref[...] = jnp.zeros_like(acc_ref)

    inv = 1.0 / _S2_SCALE
    for k, a8_ref in enumerate(a8_refs):
        @pl.when(j2 == k)
        def _(a8_ref=a8_ref):
            acc_ref[...] += jnp.dot(
                a8_ref[...], sup8_ref[...],
                preferred_element_type=jnp.float32) * inv

    @pl.when(j2 == kmin)
    def _():
        acc = acc_ref[...]
        rowsum = acc[:, f:f + 1]
        agg = acc[:, :f] + rowsum * (d_ref[0] * (1.0 / n)) + p2_ref[...]
        mix = jnp.dot(s_blk_ref[...], wc_ref[...].astype(jnp.bfloat16),
                      preferred_element_type=jnp.float32)
        h2 = jnp.maximum((1.0 - _BETA) * agg + _BETA * mix, 0.0)
        logits = jnp.dot(h2.astype(jnp.bfloat16),
                         w2_ref[...].astype(jnp.bfloat16),
                         preferred_element_type=jnp.float32) + b2_ref[...]
        m = jnp.max(logits, axis=1, keepdims=True)
        lse = m + jnp.log(jnp.sum(jnp.exp(logits - m), axis=1, keepdims=True))
        o_ref[...] = logits - lse


def kernel(x, adj, W1, b1, Wc1, Wc2, W2, b2):
    N, F = x.shape
    C = W2.shape[1]
    grid = (N // _BM,)
    nch = -(-N // _W)
    ng = N // _G
    npad = nch * _W

    V = [_W * ((_G * g) // _W) for g in range(ng)]
    kmins = [V[g] // _W for g in range(ng)]
    lens = [nch - kmins[g] for g in range(ng)]
    nsteps = sum(lens)
    maxg = [max(g for g in range(ng) if kmins[g] <= k) for k in range(nch)]
    pubs = []
    for g in range(1, ng):
        if V[g] > V[g - 1]:
            pubs.append((g * (_G // _BM), V[g - 1], V[g] - V[g - 1]))

    cumlen = [0]
    for g in range(ng):
        cumlen.append(cumlen[-1] + lens[g])

    def di(t):
        i2 = jnp.int32(0)
        for g in range(1, ng):
            i2 = i2 + (t >= cumlen[g]).astype(jnp.int32)
        return i2

    def dj(t):
        i2 = di(t)
        tstart = jnp.int32(0)
        for g in range(1, ng):
            tstart = tstart + (i2 >= g).astype(jnp.int32) * lens[g - 1]
        return (nch - 1) - (t - tstart)

    row_blk = pl.BlockSpec((_BM, F), lambda i: (i, 0))
    adj_blk = pl.BlockSpec((_BM, N), lambda i: (i, 0))
    full = lambda shape: pl.BlockSpec(shape, lambda i: (0, 0))

    h = pl.pallas_call(
        _mlp_kernel,
        grid=grid,
        in_specs=[row_blk, full((F, F)), full((1, F))],
        out_specs=row_blk,
        out_shape=jax.ShapeDtypeStruct((N, F), jnp.bfloat16),
    )(x, W1, b1.reshape(1, F))

    outs = pl.pallas_call(
        functools.partial(_layer_kernel, nch=nch, pubs=pubs),
        grid=grid,
        in_specs=[adj_blk, full((N, F)), row_blk, full((F, F))],
        out_specs=(row_blk,
                   pl.BlockSpec((_BM, F + _PAD), lambda i: (i, 0)),
                   pl.BlockSpec((1, 1, F), lambda i: (0, 0, 0)),
                   row_blk)
                  + tuple(pl.BlockSpec((_BM, _W), lambda i: (i, 0))
                          for _ in range(nch)),
        out_shape=(jax.ShapeDtypeStruct((N, F), jnp.bfloat16),
                   jax.ShapeDtypeStruct((N, F + _PAD), _F8),
                   jax.ShapeDtypeStruct((1, 1, F), jnp.float32),
                   jax.ShapeDtypeStruct((N, F), jnp.float32))
                  + tuple(jax.ShapeDtypeStruct((N, _W), _F8)
                          for _ in range(nch)),
        scratch_shapes=[pltpu.VMEM((N, F), jnp.bfloat16),
                        pltpu.VMEM((N, F), jnp.bfloat16)],
    )(adj, h, h, Wc1)
    s2, s2_8, d, p2 = outs[:4]
    a8s = outs[4:]

    s2_8p = jnp.zeros((npad, F + _PAD), _F8).at[:N].set(s2_8)

    out = pl.pallas_call(
        functools.partial(_final_kernel, nch=nch, n=N, lens=lens,
                          kmins=kmins),
        grid=(nsteps,),
        in_specs=[pl.BlockSpec((_G, _W),
                               (lambda t, mg=mg: (jnp.minimum(di(t), mg), 0)))
                  for mg in maxg]
                 + [pl.BlockSpec((_W, F + _PAD), lambda t: (dj(t), 0)),
                    pl.BlockSpec((_G, F), lambda t: (di(t), 0)),
                    pl.BlockSpec((1, 1, F), lambda t: (0, 0, 0)),
                    pl.BlockSpec((_G, F), lambda t: (di(t), 0)),
                    pl.BlockSpec((F, F), lambda t: (0, 0)),
                    pl.BlockSpec((F, C), lambda t: (0, 0)),
                    pl.BlockSpec((1, C), lambda t: (0, 0))],
        out_specs=pl.BlockSpec((_G, C), lambda t: (di(t), 0)),
        out_shape=jax.ShapeDtypeStruct((N, C), jnp.float32),
        scratch_shapes=[pltpu.VMEM((_G, F + _PAD), jnp.float32)],
    )(*a8s, s2_8p, s2, d, p2, Wc2, W2, b2.reshape(1, C))

    return out


# speedup vs baseline: 1.0190x; 1.0190x over previous
import functools

import jax
import jax.numpy as jnp
from jax.experimental import pallas as pl
from jax.experimental.pallas import tpu as pltpu

_ALPHA = 0.1
_BETA = 0.5
_BM = 400
_G = 2000
_W = 2560
_F8 = jnp.float8_e4m3fn
_S2_SCALE = 1.0 / 64.0
_PAD = 8


def _mlp_kernel(x_ref, w_ref, b_ref, o_ref):
    h = jnp.dot(x_ref[...].astype(jnp.bfloat16),
                w_ref[...].astype(jnp.bfloat16),
                preferred_element_type=jnp.float32)
    o_ref[...] = jnp.maximum(h + b_ref[...], 0.0).astype(jnp.bfloat16)


def _layer_kernel(adj_ref, sup_ref, h_blk_ref, wc_ref,
                  o_ref, s8_ref, d_ref, p2_ref, a8_ref,
                  s2vis_ref, stage_ref, *, pubs):
    i = pl.program_id(0)

    @pl.when(i == 0)
    def _():
        s2vis_ref[...] = jnp.zeros_like(s2vis_ref)

    for step, start, size in pubs:
        @pl.when(i == step)
        def _(start=start, size=size):
            s2vis_ref[start:start + size, :] = stage_ref[start:start + size, :]

    a = adj_ref[...]
    a8 = a.astype(_F8)
    npad = a8_ref.shape[1]
    n = a.shape[1]
    if npad > n:
        a8 = jnp.concatenate(
            [a8, jnp.zeros((a.shape[0], npad - n), _F8)], axis=1)
    a8_ref[...] = a8

    ab = a.astype(jnp.bfloat16)
    agg = jnp.dot(ab, sup_ref[...], preferred_element_type=jnp.float32)
    p2_ref[...] = jnp.dot(ab, s2vis_ref[...],
                          preferred_element_type=jnp.float32)
    mix = jnp.dot(h_blk_ref[...], wc_ref[...].astype(jnp.bfloat16),
                  preferred_element_type=jnp.float32)
    out = jnp.maximum((1.0 - _BETA) * agg + _BETA * mix, 0.0)
    s2 = ((1.0 - _ALPHA) * out
          + _ALPHA * h_blk_ref[...].astype(jnp.float32))
    s2_bf = s2.astype(jnp.bfloat16)
    o_ref[...] = s2_bf
    stage_ref[pl.ds(i * _BM, _BM), :] = s2_bf

    s8 = (s2 * _S2_SCALE).astype(_F8)
    probe = jnp.where(
        jax.lax.broadcasted_iota(jnp.int32, (s2.shape[0], _PAD), 1) == 0,
        jnp.float32(_S2_SCALE), 0.0).astype(_F8)
    s8_ref[...] = jnp.concatenate([s8, probe], axis=1)

    delta = s2 - s8.astype(jnp.float32) * (1.0 / _S2_SCALE)
    dcol = jnp.sum(delta, axis=0, keepdims=True)[None]

    @pl.when(i == 0)
    def _():
        d_ref[...] = dcol

    @pl.when(i > 0)
    def _():
        d_ref[...] += dcol


def _final_kernel(a8_ref, sup8_ref, s_blk_ref, d_ref, p2_ref,
                  wc_ref, w2_ref, b2_ref, o_ref, acc_ref,
                  *, nch, n, lens, kmins):
    f = s_blk_ref.shape[1]
    t = pl.program_id(0)

    i2 = jnp.int32(0)
    cum = 0
    for g in range(len(lens) - 1):
        cum += lens[g]
        i2 = i2 + (t >= cum).astype(jnp.int32)
    tstart = jnp.int32(0)
    for g in range(1, len(lens)):
        tstart = tstart + (i2 >= g).astype(jnp.int32) * lens[g - 1]
    j2 = (nch - 1) - (t - tstart)
    kmin = jnp.int32(0)
    for g in range(1, len(lens)):
        kmin = kmin + (i2 >= g).astype(jnp.int32) * (kmins[g] - kmins[g - 1])

    inv = 1.0 / _S2_SCALE
    contrib = jnp.dot(a8_ref[...], sup8_ref[...],
                      preferred_element_type=jnp.float32) * inv

    @pl.when(j2 == nch - 1)
    def _():
        acc_ref[...] = contrib

    @pl.when(j2 < nch - 1)
    def _():
        acc_ref[...] += contrib

    @pl.when(j2 == kmin)
    def _():
        acc = acc_ref[...]
        rowsum = acc[:, f:f + 1]
        agg = acc[:, :f] + rowsum * (d_ref[0] * (1.0 / n)) + p2_ref[...]
        mix = jnp.dot(s_blk_ref[...], wc_ref[...].astype(jnp.bfloat16),
                      preferred_element_type=jnp.float32)
        h2 = jnp.maximum((1.0 - _BETA) * agg + _BETA * mix, 0.0)
        logits = jnp.dot(h2.astype(jnp.bfloat16),
                         w2_ref[...].astype(jnp.bfloat16),
                         preferred_element_type=jnp.float32) + b2_ref[...]
        m = jnp.max(logits, axis=1, keepdims=True)
        lse = m + jnp.log(jnp.sum(jnp.exp(logits - m), axis=1, keepdims=True))
        o_ref[...] = logits - lse


def kernel(x, adj, W1, b1, Wc1, Wc2, W2, b2):
    N, F = x.shape
    C = W2.shape[1]
    grid = (N // _BM,)
    nch = -(-N // _W)
    ng = N // _G
    npad = nch * _W

    V = [_W * ((_G * g) // _W) for g in range(ng)]
    kmins = [V[g] // _W for g in range(ng)]
    lens = [nch - kmins[g] for g in range(ng)]
    nsteps = sum(lens)
    maxg = [max(g for g in range(ng) if kmins[g] <= k) for k in range(nch)]
    pubs = []
    for g in range(1, ng):
        if V[g] > V[g - 1]:
            pubs.append((g * (_G // _BM), V[g - 1], V[g] - V[g - 1]))

    cumlen = [0]
    for g in range(ng):
        cumlen.append(cumlen[-1] + lens[g])

    def di(t):
        i2 = jnp.int32(0)
        for g in range(1, ng):
            i2 = i2 + (t >= cumlen[g]).astype(jnp.int32)
        return i2

    def dj(t):
        i2 = di(t)
        tstart = jnp.int32(0)
        for g in range(1, ng):
            tstart = tstart + (i2 >= g).astype(jnp.int32) * lens[g - 1]
        return (nch - 1) - (t - tstart)

    row_blk = pl.BlockSpec((_BM, F), lambda i: (i, 0))
    adj_blk = pl.BlockSpec((_BM, N), lambda i: (i, 0))
    full = lambda shape: pl.BlockSpec(shape, lambda i: (0, 0))

    h = pl.pallas_call(
        _mlp_kernel,
        grid=grid,
        in_specs=[row_blk, full((F, F)), full((1, F))],
        out_specs=row_blk,
        out_shape=jax.ShapeDtypeStruct((N, F), jnp.bfloat16),
    )(x, W1, b1.reshape(1, F))

    s2, s2_8, d, p2, adj8 = pl.pallas_call(
        functools.partial(_layer_kernel, pubs=pubs),
        grid=grid,
        in_specs=[adj_blk, full((N, F)), row_blk, full((F, F))],
        out_specs=(row_blk,
                   pl.BlockSpec((_BM, F + _PAD), lambda i: (i, 0)),
                   pl.BlockSpec((1, 1, F), lambda i: (0, 0, 0)),
                   row_blk,
                   pl.BlockSpec((_BM, npad), lambda i: (i, 0))),
        out_shape=(jax.ShapeDtypeStruct((N, F), jnp.bfloat16),
                   jax.ShapeDtypeStruct((N, F + _PAD), _F8),
                   jax.ShapeDtypeStruct((1, 1, F), jnp.float32),
                   jax.ShapeDtypeStruct((N, F), jnp.float32),
                   jax.ShapeDtypeStruct((N, npad), _F8)),
        scratch_shapes=[pltpu.VMEM((N, F), jnp.bfloat16),
                        pltpu.VMEM((N, F), jnp.bfloat16)],
    )(adj, h, h, Wc1)

    s2_8p = jnp.zeros((npad, F + _PAD), _F8).at[:N].set(s2_8)

    out = pl.pallas_call(
        functools.partial(_final_kernel, nch=nch, n=N, lens=lens,
                          kmins=kmins),
        grid=(nsteps,),
        in_specs=[pl.BlockSpec((_G, _W), lambda t: (di(t), dj(t))),
                  pl.BlockSpec((_W, F + _PAD), lambda t: (dj(t), 0)),
                  pl.BlockSpec((_G, F), lambda t: (di(t), 0)),
                  pl.BlockSpec((1, 1, F), lambda t: (0, 0, 0)),
                  pl.BlockSpec((_G, F), lambda t: (di(t), 0)),
                  pl.BlockSpec((F, F), lambda t: (0, 0)),
                  pl.BlockSpec((F, C), lambda t: (0, 0)),
                  pl.BlockSpec((1, C), lambda t: (0, 0))],
        out_specs=pl.BlockSpec((_G, C), lambda t: (di(t), 0)),
        out_shape=jax.ShapeDtypeStruct((N, C), jnp.float32),
        scratch_shapes=[pltpu.VMEM((_G, F + _PAD), jnp.float32)],
    )(adj8, s2_8p, s2, d, p2, Wc2, W2, b2.reshape(1, C))

    return out


# speedup vs baseline: 1.1920x; 1.1697x over previous
import jax
import jax.numpy as jnp
from jax.experimental import pallas as pl
from jax.experimental.pallas import tpu as pltpu

_ALPHA = 0.1
_BETA = 0.5
_BM = 400
_BM2 = 1000
_F8 = jnp.float8_e4m3fn
_S2_SCALE = 1.0 / 64.0
_PAD = 8


def _layer_kernel(x_ref, w1_ref, b1_ref, adj_ref, wc_ref,
                  o_ref, a8_ref, s8_ref, d_ref, h_ref):
    i = pl.program_id(0)

    @pl.when(i == 0)
    def _():
        hv = jnp.dot(x_ref[...].astype(jnp.bfloat16),
                     w1_ref[...].astype(jnp.bfloat16),
                     preferred_element_type=jnp.float32)
        h_ref[...] = jnp.maximum(hv + b1_ref[...], 0.0).astype(jnp.bfloat16)

    a = adj_ref[...]
    a8_ref[...] = a.astype(_F8)
    h_blk = h_ref[pl.ds(i * a.shape[0], a.shape[0]), :]
    agg = jnp.dot(a.astype(jnp.bfloat16), h_ref[...],
                  preferred_element_type=jnp.float32)
    mix = jnp.dot(h_blk, wc_ref[...].astype(jnp.bfloat16),
                  preferred_element_type=jnp.float32)
    out = jnp.maximum((1.0 - _BETA) * agg + _BETA * mix, 0.0)
    s2 = ((1.0 - _ALPHA) * out
          + _ALPHA * h_blk.astype(jnp.float32))
    o_ref[...] = s2.astype(jnp.bfloat16)

    s8 = (s2 * _S2_SCALE).astype(_F8)
    bm = s2.shape[0]
    probe = jnp.where(
        jax.lax.broadcasted_iota(jnp.int32, (bm, _PAD), 1) == 0,
        jnp.float32(_S2_SCALE), 0.0).astype(_F8)
    s8_ref[...] = jnp.concatenate([s8, probe], axis=1)

    delta = s2 - s8.astype(jnp.float32) * (1.0 / _S2_SCALE)
    dcol = jnp.sum(delta, axis=0, keepdims=True)
    i = pl.program_id(0)

    @pl.when(i == 0)
    def _():
        d_ref[...] = dcol

    @pl.when(i > 0)
    def _():
        d_ref[...] += dcol


def _final_kernel(a8_ref, sup8_ref, s_blk_ref, d_ref, wc_ref, w2_ref, b2_ref,
                  o_ref):
    f = s_blk_ref.shape[1]
    agg_ext = jnp.dot(a8_ref[...], sup8_ref[...],
                      preferred_element_type=jnp.float32) * (1.0 / _S2_SCALE)
    rowsum = agg_ext[:, f:f + 1]
    n = a8_ref.shape[1]
    agg = agg_ext[:, :f] + rowsum * (d_ref[...] * (1.0 / n))
    mix = jnp.dot(s_blk_ref[...], wc_ref[...].astype(jnp.bfloat16),
                  preferred_element_type=jnp.float32)
    h2 = jnp.maximum((1.0 - _BETA) * agg + _BETA * mix, 0.0)
    logits = jnp.dot(h2.astype(jnp.bfloat16), w2_ref[...].astype(jnp.bfloat16),
                     preferred_element_type=jnp.float32) + b2_ref[...]
    m = jnp.max(logits, axis=1, keepdims=True)
    lse = m + jnp.log(jnp.sum(jnp.exp(logits - m), axis=1, keepdims=True))
    o_ref[...] = logits - lse


def kernel(x, adj, W1, b1, Wc1, Wc2, W2, b2):
    N, F = x.shape
    C = W2.shape[1]
    grid = (N // _BM,)

    row_blk = pl.BlockSpec((_BM, F), lambda i: (i, 0))
    adj_blk = pl.BlockSpec((_BM, N), lambda i: (i, 0))
    full = lambda shape: pl.BlockSpec(shape, lambda i: (0, 0))

    s2, adj8, s2_8, d = pl.pallas_call(
        _layer_kernel,
        grid=grid,
        in_specs=[full((N, F)), full((F, F)), full((1, F)),
                  adj_blk, full((F, F))],
        out_specs=(row_blk, adj_blk,
                   pl.BlockSpec((_BM, F + _PAD), lambda i: (i, 0)),
                   full((1, F))),
        out_shape=(jax.ShapeDtypeStruct((N, F), jnp.bfloat16),
                   jax.ShapeDtypeStruct((N, N), _F8),
                   jax.ShapeDtypeStruct((N, F + _PAD), _F8),
                   jax.ShapeDtypeStruct((1, F), jnp.float32)),
        scratch_shapes=[pltpu.VMEM((N, F), jnp.bfloat16)],
    )(x, W1, b1.reshape(1, F), adj, Wc1)

    out = pl.pallas_call(
        _final_kernel,
        grid=(N // _BM2,),
        in_specs=[pl.BlockSpec((_BM2, N), lambda i: (i, 0)),
                  full((N, F + _PAD)),
                  pl.BlockSpec((_BM2, F), lambda i: (i, 0)),
                  full((1, F)),
                  full((F, F)), full((F, C)), full((1, C))],
        out_specs=pl.BlockSpec((_BM2, C), lambda i: (i, 0)),
        out_shape=jax.ShapeDtypeStruct((N, C), jnp.float32),
    )(adj8, s2_8, s2, d, Wc2, W2, b2.reshape(1, C))

    return out


# speedup vs baseline: 1.1962x; 1.0035x over previous
import jax
import jax.numpy as jnp
from jax.experimental import pallas as pl
from jax.experimental.pallas import tpu as pltpu

_ALPHA = 0.1
_BETA = 0.5
_BM = 200
_BM2 = 1000
_F8 = jnp.float8_e4m3fn
_S2_SCALE = 1.0 / 64.0
_PAD = 8


def _layer_kernel(x_ref, w1_ref, b1_ref, adj_ref, wc_ref,
                  o_ref, a8_ref, s8_ref, d_ref, h_ref):
    i = pl.program_id(0)

    @pl.when(i == 0)
    def _():
        hv = jnp.dot(x_ref[...].astype(jnp.bfloat16),
                     w1_ref[...].astype(jnp.bfloat16),
                     preferred_element_type=jnp.float32)
        h_ref[...] = jnp.maximum(hv + b1_ref[...], 0.0).astype(jnp.bfloat16)

    a = adj_ref[...]
    a8_ref[...] = a.astype(_F8)
    h_blk = h_ref[pl.ds(i * a.shape[0], a.shape[0]), :]
    agg = jnp.dot(a.astype(jnp.bfloat16), h_ref[...],
                  preferred_element_type=jnp.float32)
    mix = jnp.dot(h_blk, wc_ref[...].astype(jnp.bfloat16),
                  preferred_element_type=jnp.float32)
    out = jnp.maximum((1.0 - _BETA) * agg + _BETA * mix, 0.0)
    s2 = ((1.0 - _ALPHA) * out
          + _ALPHA * h_blk.astype(jnp.float32))
    o_ref[...] = s2.astype(jnp.bfloat16)

    s8 = (s2 * _S2_SCALE).astype(_F8)
    bm = s2.shape[0]
    probe = jnp.where(
        jax.lax.broadcasted_iota(jnp.int32, (bm, _PAD), 1) == 0,
        jnp.float32(_S2_SCALE), 0.0).astype(_F8)
    s8_ref[...] = jnp.concatenate([s8, probe], axis=1)

    delta = s2 - s8.astype(jnp.float32) * (1.0 / _S2_SCALE)
    dcol = jnp.sum(delta, axis=0, keepdims=True)
    i = pl.program_id(0)

    @pl.when(i == 0)
    def _():
        d_ref[...] = dcol

    @pl.when(i > 0)
    def _():
        d_ref[...] += dcol


def _final_kernel(a8_ref, sup8_ref, s_blk_ref, d_ref, wc_ref, w2_ref, b2_ref,
                  o_ref):
    f = s_blk_ref.shape[1]
    agg_ext = jnp.dot(a8_ref[...], sup8_ref[...],
                      preferred_element_type=jnp.float32) * (1.0 / _S2_SCALE)
    rowsum = agg_ext[:, f:f + 1]
    n = a8_ref.shape[1]
    agg = agg_ext[:, :f] + rowsum * (d_ref[...] * (1.0 / n))
    mix = jnp.dot(s_blk_ref[...], wc_ref[...].astype(jnp.bfloat16),
                  preferred_element_type=jnp.float32)
    h2 = jnp.maximum((1.0 - _BETA) * agg + _BETA * mix, 0.0)
    logits = jnp.dot(h2.astype(jnp.bfloat16), w2_ref[...].astype(jnp.bfloat16),
                     preferred_element_type=jnp.float32) + b2_ref[...]
    m = jnp.max(logits, axis=1, keepdims=True)
    lse = m + jnp.log(jnp.sum(jnp.exp(logits - m), axis=1, keepdims=True))
    o_ref[...] = logits - lse


def kernel(x, adj, W1, b1, Wc1, Wc2, W2, b2):
    N, F = x.shape
    C = W2.shape[1]
    grid = (N // _BM,)

    row_blk = pl.BlockSpec((_BM, F), lambda i: (i, 0))
    adj_blk = pl.BlockSpec((_BM, N), lambda i: (i, 0))
    full = lambda shape: pl.BlockSpec(shape, lambda i: (0, 0))

    s2, adj8, s2_8, d = pl.pallas_call(
        _layer_kernel,
        grid=grid,
        in_specs=[full((N, F)), full((F, F)), full((1, F)),
                  adj_blk, full((F, F))],
        out_specs=(row_blk, adj_blk,
                   pl.BlockSpec((_BM, F + _PAD), lambda i: (i, 0)),
                   full((1, F))),
        out_shape=(jax.ShapeDtypeStruct((N, F), jnp.bfloat16),
                   jax.ShapeDtypeStruct((N, N), _F8),
                   jax.ShapeDtypeStruct((N, F + _PAD), _F8),
                   jax.ShapeDtypeStruct((1, F), jnp.float32)),
        scratch_shapes=[pltpu.VMEM((N, F), jnp.bfloat16)],
    )(x, W1, b1.reshape(1, F), adj, Wc1)

    out = pl.pallas_call(
        _final_kernel,
        grid=(N // _BM2,),
        in_specs=[pl.BlockSpec((_BM2, N), lambda i: (i, 0)),
                  full((N, F + _PAD)),
                  pl.BlockSpec((_BM2, F), lambda i: (i, 0)),
                  full((1, F)),
                  full((F, F)), full((F, C)), full((1, C))],
        out_specs=pl.BlockSpec((_BM2, C), lambda i: (i, 0)),
        out_shape=jax.ShapeDtypeStruct((N, C), jnp.float32),
    )(adj8, s2_8, s2, d, Wc2, W2, b2.reshape(1, C))

    return out
